# baseline (speedup 1.0000x reference)
import jax
import jax.numpy as jnp
from jax import lax
from jax.experimental import pallas as pl
from jax.experimental.pallas import tpu as pltpu
from jax.experimental.pallas import tpu_sc as plsc

_B = 16384
_D = 16
_R = 8
_NW = 32
_NC = 2
_BPW = _B // _NW
_CH = 128
_NCH = _BPW // _CH
_V = 16


def _gf_body(i_hbm, j_hbm, emb_hbm, out_hbm, idx_i, idx_j, q_i, q_j,
             buf_a0, buf_b0, buf_a1, buf_b1, out_v, sem0, sem1):
    wid = lax.axis_index("s") * _NC + lax.axis_index("c")
    base = wid * _BPW
    pltpu.sync_copy(i_hbm.at[pl.ds(base, _BPW)], idx_i)
    pltpu.sync_copy(j_hbm.at[pl.ds(base, _BPW)], idx_j)

    for v in range(_BPW // _V):
        sl = pl.ds(v * _V, _V)
        q_i[sl] = lax.shift_right_logical(idx_i[sl], 3)
        q_j[sl] = lax.shift_right_logical(idx_j[sl], 3)

    bufs = ((buf_a0, buf_b0, sem0), (buf_a1, buf_b1, sem1))

    def issue(c, p):
        buf_a, buf_b, sem = bufs[p]
        sl = pl.ds(c * _CH, _CH)
        pltpu.async_copy(emb_hbm.at[q_i.at[sl]], buf_a, sem)
        pltpu.async_copy(emb_hbm.at[q_j.at[sl]], buf_b, sem)

    def wait_and_compute(c, p):
        buf_a, buf_b, sem = bufs[p]
        pltpu.make_async_copy(emb_hbm.at[pl.ds(0, _CH)], buf_a, sem).wait()
        pltpu.make_async_copy(emb_hbm.at[pl.ds(0, _CH)], buf_b, sem).wait()
        for s0 in range(0, _CH, _V):
            sl = pl.ds(c * _CH + s0, _V)
            rows = s0 + lax.iota(jnp.int32, _V)
            col_a = (idx_i[sl] & 7) * _D
            col_b = (idx_j[sl] & 7) * _D
            acc = plsc.load_gather(buf_a, [rows, col_a]) * \
                plsc.load_gather(buf_b, [rows, col_b])
            for d in range(1, _D):
                acc = acc + plsc.load_gather(buf_a, [rows, col_a + d]) * \
                    plsc.load_gather(buf_b, [rows, col_b + d])
            out_v[sl] = 1.0 / (1.0 + jnp.exp(-acc))

    issue(0, 0)
    for c in range(_NCH):
        if c + 1 < _NCH:
            issue(c + 1, (c + 1) % 2)
        wait_and_compute(c, c % 2)

    pltpu.sync_copy(out_v, out_hbm.at[pl.ds(base, _BPW)])


@jax.jit
def _gf(i, j, emb):
    emb128 = emb.reshape(125000, 128)
    return pl.kernel(
        _gf_body,
        out_type=jax.ShapeDtypeStruct((_B,), jnp.float32),
        mesh=plsc.VectorSubcoreMesh(core_axis_name="c", subcore_axis_name="s"),
        scratch_types=[
            pltpu.VMEM((_BPW,), jnp.int32),
            pltpu.VMEM((_BPW,), jnp.int32),
            pltpu.VMEM((_BPW,), jnp.int32),
            pltpu.VMEM((_BPW,), jnp.int32),
            pltpu.VMEM((_CH, 128), jnp.float32),
            pltpu.VMEM((_CH, 128), jnp.float32),
            pltpu.VMEM((_CH, 128), jnp.float32),
            pltpu.VMEM((_CH, 128), jnp.float32),
            pltpu.VMEM((_BPW,), jnp.float32),
            pltpu.SemaphoreType.DMA,
            pltpu.SemaphoreType.DMA,
        ],
        compiler_params=pltpu.CompilerParams(needs_layout_passes=False),
    )(i, j, emb128)


def kernel(i, j, emb):
    return _gf(i, j, emb)
